# baseline (speedup 1.0000x reference)
import jax
import jax.numpy as jnp
from jax.experimental import pallas as pl
from jax.experimental.pallas import tpu as pltpu

_ROWS = 101
_ALIGNED = 96
_TAIL = _ROWS - _ALIGNED
_CHUNK = 128
_NSEM = 8


def _body(emb_ref, out_ref, mask_ref, abuf, tbuf, mbuf, asem, tsem, msem):
    abuf[...] = jnp.broadcast_to(emb_ref[:_ALIGNED][None, :, :], abuf.shape)
    tbuf[...] = jnp.broadcast_to(emb_ref[_ALIGNED:][None, :, :], tbuf.shape)
    mbuf[...] = jnp.ones(mbuf.shape, jnp.float32)
    nchunks = out_ref.shape[0] // _CHUNK
    copies = []
    for i in range(nchunks):
        if i >= _NSEM:
            for c in copies[i - _NSEM]:
                c.wait()
        b = pl.ds(i * _CHUNK, _CHUNK)
        a = pltpu.make_async_copy(
            abuf, out_ref.at[b, pl.ds(0, _ALIGNED), :], asem.at[i % _NSEM])
        t = pltpu.make_async_copy(
            tbuf, out_ref.at[b, pl.ds(_ALIGNED, _TAIL), :], tsem.at[i % _NSEM])
        m = pltpu.make_async_copy(mbuf, mask_ref.at[b], msem.at[i % _NSEM])
        a.start()
        t.start()
        m.start()
        copies.append((a, t, m))
    for i in range(max(0, nchunks - _NSEM), nchunks):
        for c in copies[i]:
            c.wait()


def kernel(feature_map, key, embedding):
    del key
    batch = feature_map.shape[1]
    embed_dim = embedding.shape[1]
    emb = embedding[:_ROWS]

    tiled, mask = pl.pallas_call(
        _body,
        in_specs=[pl.BlockSpec(memory_space=pltpu.MemorySpace.VMEM)],
        out_specs=[
            pl.BlockSpec(memory_space=pl.ANY),
            pl.BlockSpec(memory_space=pl.ANY),
        ],
        out_shape=[
            jax.ShapeDtypeStruct((batch, _ROWS, embed_dim), jnp.float32),
            jax.ShapeDtypeStruct((batch, _ROWS), jnp.float32),
        ],
        scratch_shapes=[
            pltpu.VMEM((_CHUNK, _ALIGNED, embed_dim), jnp.float32),
            pltpu.VMEM((_CHUNK, _TAIL, embed_dim), jnp.float32),
            pltpu.VMEM((_CHUNK, _ROWS), jnp.float32),
            pltpu.SemaphoreType.DMA((_NSEM,)),
            pltpu.SemaphoreType.DMA((_NSEM,)),
            pltpu.SemaphoreType.DMA((_NSEM,)),
        ],
    )(emb)
    return (tiled, mask)
